# Initial kernel scaffold; baseline (speedup 1.0000x reference)
import jax
import jax.numpy as jnp
from jax.experimental import pallas as pl


def kernel(x, edge_index, W1_l, b1_l, W1_r, W2_l, b2_l, W2_r):
    raise NotImplementedError("write your pallas kernel here")


# speedup vs baseline: 5.5131x; 5.5131x over previous
import functools

import jax
import jax.numpy as jnp
from jax import lax
from jax.experimental import pallas as pl
from jax.experimental.pallas import tpu as pltpu
from jax.experimental.pallas import tpu_sc as plsc

N = 10000
E = 320000
D = 128
DH = D // 2

NC = 2
NS = 16
EPW = E // NS
CHUNK = 80
NCH = EPW // CHUNK
RPAD = 10240
RPS = RPAD // NS

_mesh = plsc.VectorSubcoreMesh(core_axis_name="c", subcore_axis_name="s")


def _make_agg(with_counts: bool):
  out_type = [jax.ShapeDtypeStruct((NC, RPAD, DH), jnp.float32)]
  scratch = [
      pltpu.VMEM((NCH, CHUNK), jnp.int32),
      pltpu.VMEM((NCH, CHUNK), jnp.int32),
      pltpu.VMEM((CHUNK, DH), jnp.float32),
      pltpu.VMEM_SHARED((RPAD, DH), jnp.float32),
      pltpu.SemaphoreType.DMA,
  ]
  if with_counts:
    out_type.append(jax.ShapeDtypeStruct((RPAD, 16), jnp.float32))
    scratch.append(pltpu.VMEM((CHUNK, 16), jnp.float32))
    scratch.append(pltpu.VMEM_SHARED((RPAD, 16), jnp.float32))

  @functools.partial(pl.kernel, mesh=_mesh, out_type=out_type,
                     scratch_types=scratch,
                     compiler_params=pltpu.CompilerParams(
                         use_tc_tiling_on_sc=False))
  def agg(*refs):
    if with_counts:
      (feat2, src3, dst3, zrow, zcnt, ones_h,
       agg_out, cnt_out, src_v, dst_v, rows_v, acc_s, sem,
       ones_v, cnt_s) = refs
    else:
      (feat2, src3, dst3, zrow,
       agg_out, src_v, dst_v, rows_v, acc_s, sem) = refs
    cid = lax.axis_index("c")
    sid = lax.axis_index("s")
    row0 = sid * RPS
    pltpu.sync_copy(zrow, acc_s.at[pl.ds(row0, RPS)])
    pltpu.sync_copy(src3.at[sid], src_v)
    pltpu.sync_copy(dst3.at[sid], dst_v)
    if with_counts:
      @pl.when(cid == 0)
      def _():
        pltpu.sync_copy(zcnt, cnt_s.at[pl.ds(row0, RPS)])
        pltpu.sync_copy(ones_h, ones_v)
    plsc.subcore_barrier()

    def body(i, carry):
      pltpu.async_copy(feat2.at[cid].at[src_v.at[i]], rows_v, sem).wait()
      pltpu.sync_copy(rows_v, acc_s.at[dst_v.at[i]], add=True)
      if with_counts:
        @pl.when(cid == 0)
        def _():
          pltpu.sync_copy(ones_v, cnt_s.at[dst_v.at[i]], add=True)
      return carry

    lax.fori_loop(0, NCH, body, 0)
    plsc.subcore_barrier()
    pltpu.sync_copy(acc_s.at[pl.ds(row0, RPS)],
                    agg_out.at[cid, pl.ds(row0, RPS)])
    if with_counts:
      @pl.when(cid == 0)
      def _():
        pltpu.sync_copy(cnt_s.at[pl.ds(row0, RPS)],
                        cnt_out.at[pl.ds(row0, RPS)])

  return agg


_agg_counts = _make_agg(True)
_agg_plain = _make_agg(False)

_R = 1000


def _make_dense(relu: bool, split_out: bool):
  def body(agg_ref, cnt_ref, x_ref, wl_ref, b_ref, wr_ref, *o_refs):
    inv = 1.0 / jnp.maximum(cnt_ref[:, 0:1], 1.0)
    y = (jnp.dot(agg_ref[0] * inv, wl_ref[0:DH, :],
                 preferred_element_type=jnp.float32)
         + jnp.dot(agg_ref[1] * inv, wl_ref[DH:D, :],
                   preferred_element_type=jnp.float32)
         + b_ref[...]
         + jnp.dot(x_ref[...], wr_ref[...], preferred_element_type=jnp.float32))
    if relu:
      y = jnp.maximum(y, 0.0)
    o_refs[0][...] = y
    if split_out:
      o_refs[1][0] = y[:, 0:DH]
      o_refs[1][1] = y[:, DH:D]

  out_shape = [jax.ShapeDtypeStruct((N, D), jnp.float32)]
  out_specs = [pl.BlockSpec((_R, D), lambda i: (i, 0))]
  if split_out:
    out_shape.append(jax.ShapeDtypeStruct((NC, N, DH), jnp.float32))
    out_specs.append(pl.BlockSpec((NC, _R, DH), lambda i: (0, i, 0)))

  return pl.pallas_call(
      body,
      grid=(N // _R,),
      in_specs=[
          pl.BlockSpec((NC, _R, DH), lambda i: (0, i, 0)),
          pl.BlockSpec((_R, 16), lambda i: (i, 0)),
          pl.BlockSpec((_R, D), lambda i: (i, 0)),
          pl.BlockSpec((D, D), lambda i: (0, 0)),
          pl.BlockSpec((1, D), lambda i: (0, 0)),
          pl.BlockSpec((D, D), lambda i: (0, 0)),
      ],
      out_specs=out_specs,
      out_shape=out_shape,
  )


_dense_relu_split = _make_dense(True, True)
_dense_lin = _make_dense(False, False)


def kernel(x, edge_index, W1_l, b1_l, W1_r, W2_l, b2_l, W2_r):
  src = edge_index[0].astype(jnp.int32).reshape(NS, NCH, CHUNK)
  dst = edge_index[1].astype(jnp.int32).reshape(NS, NCH, CHUNK)
  xs = jnp.moveaxis(x.reshape(N, NC, DH), 1, 0)
  zrow = jnp.zeros((RPS, DH), jnp.float32)
  zcnt = jnp.zeros((RPS, 16), jnp.float32)
  ones = jnp.ones((CHUNK, 16), jnp.float32)
  b1 = b1_l.reshape(1, D)
  b2 = b2_l.reshape(1, D)

  agg1, cnt = _agg_counts(xs, src, dst, zrow, zcnt, ones)
  h, hs = _dense_relu_split(agg1, cnt, x, W1_l, b1, W1_r)
  agg2, = _agg_plain(hs, src, dst, zrow)
  out, = _dense_lin(agg2, cnt, h, W2_l, b2, W2_r)
  return out


# speedup vs baseline: 8.9925x; 1.6311x over previous
import functools

import jax
import jax.numpy as jnp
from jax import lax
from jax.experimental import pallas as pl
from jax.experimental.pallas import tpu as pltpu
from jax.experimental.pallas import tpu_sc as plsc

N = 10000
E = 320000
D = 128
DH = D // 2

NC = 2
NS = 16
EPW = E // NS
CHUNK = 80
NCH = EPW // CHUNK
RPAD = 10240
RPS = RPAD // NS

_mesh = plsc.VectorSubcoreMesh(core_axis_name="c", subcore_axis_name="s")


def _make_agg(with_counts: bool):
  out_type = [jax.ShapeDtypeStruct((NC, RPAD, DH), jnp.float32)]
  scratch = [
      pltpu.VMEM((NCH, CHUNK), jnp.int32),
      pltpu.VMEM((NCH, CHUNK), jnp.int32),
      pltpu.VMEM((CHUNK, DH), jnp.float32),
      pltpu.VMEM((CHUNK, DH), jnp.float32),
      pltpu.VMEM_SHARED((RPAD, DH), jnp.float32),
      pltpu.SemaphoreType.DMA,
      pltpu.SemaphoreType.DMA,
  ]
  if with_counts:
    out_type.append(jax.ShapeDtypeStruct((RPAD, 16), jnp.float32))
    scratch.append(pltpu.VMEM((CHUNK, 16), jnp.float32))
    scratch.append(pltpu.VMEM_SHARED((RPAD, 16), jnp.float32))

  @functools.partial(pl.kernel, mesh=_mesh, out_type=out_type,
                     scratch_types=scratch,
                     compiler_params=pltpu.CompilerParams(
                         use_tc_tiling_on_sc=False))
  def agg(*refs):
    if with_counts:
      (feat2, src3, dst3, zrow, zcnt, ones_h,
       agg_out, cnt_out, src_v, dst_v, rows0, rows1, acc_s, sem0, sem1,
       ones_v, cnt_s) = refs
    else:
      (feat2, src3, dst3, zrow,
       agg_out, src_v, dst_v, rows0, rows1, acc_s, sem0, sem1) = refs
    cid = lax.axis_index("c")
    sid = lax.axis_index("s")
    row0 = sid * RPS
    pltpu.sync_copy(zrow, acc_s.at[pl.ds(row0, RPS)])
    pltpu.sync_copy(src3.at[sid], src_v)
    pltpu.sync_copy(dst3.at[sid], dst_v)
    if with_counts:
      @pl.when(cid == 0)
      def _():
        pltpu.sync_copy(zcnt, cnt_s.at[pl.ds(row0, RPS)])
        pltpu.sync_copy(ones_h, ones_v)
    plsc.subcore_barrier()

    def gather(i, buf, sem):
      return pltpu.async_copy(feat2.at[cid].at[src_v.at[i]], buf, sem)

    def gwait(i, buf, sem):
      pltpu.make_async_copy(feat2.at[cid].at[src_v.at[i]], buf, sem).wait()

    def scatter(i, buf):
      pltpu.sync_copy(buf, acc_s.at[dst_v.at[i]], add=True)
      if with_counts:
        @pl.when(cid == 0)
        def _():
          pltpu.sync_copy(ones_v, cnt_s.at[dst_v.at[i]], add=True)

    npair = NCH // 2
    gather(0, rows0, sem0)

    def body(j, carry):
      i0 = 2 * j
      gather(i0 + 1, rows1, sem1)
      gwait(i0, rows0, sem0)
      scatter(i0, rows0)

      @pl.when(j < npair - 1)
      def _():
        gather(i0 + 2, rows0, sem0)
      gwait(i0 + 1, rows1, sem1)
      scatter(i0 + 1, rows1)
      return carry

    lax.fori_loop(0, npair, body, 0)
    plsc.subcore_barrier()
    pltpu.sync_copy(acc_s.at[pl.ds(row0, RPS)],
                    agg_out.at[cid, pl.ds(row0, RPS)])
    if with_counts:
      @pl.when(cid == 0)
      def _():
        pltpu.sync_copy(cnt_s.at[pl.ds(row0, RPS)],
                        cnt_out.at[pl.ds(row0, RPS)])

  return agg


_agg_counts = _make_agg(True)
_agg_plain = _make_agg(False)

_R = 1000


def _make_dense(relu: bool, split_out: bool):
  def body(agg_ref, cnt_ref, x_ref, wl_ref, b_ref, wr_ref, *o_refs):
    inv = 1.0 / jnp.maximum(cnt_ref[:, 0:1], 1.0)
    y = (jnp.dot(agg_ref[0] * inv, wl_ref[0:DH, :],
                 preferred_element_type=jnp.float32)
         + jnp.dot(agg_ref[1] * inv, wl_ref[DH:D, :],
                   preferred_element_type=jnp.float32)
         + b_ref[...]
         + jnp.dot(x_ref[...], wr_ref[...], preferred_element_type=jnp.float32))
    if relu:
      y = jnp.maximum(y, 0.0)
    o_refs[0][...] = y
    if split_out:
      o_refs[1][0] = y[:, 0:DH]
      o_refs[1][1] = y[:, DH:D]

  out_shape = [jax.ShapeDtypeStruct((N, D), jnp.float32)]
  out_specs = [pl.BlockSpec((_R, D), lambda i: (i, 0))]
  if split_out:
    out_shape.append(jax.ShapeDtypeStruct((NC, N, DH), jnp.float32))
    out_specs.append(pl.BlockSpec((NC, _R, DH), lambda i: (0, i, 0)))

  return pl.pallas_call(
      body,
      grid=(N // _R,),
      in_specs=[
          pl.BlockSpec((NC, _R, DH), lambda i: (0, i, 0)),
          pl.BlockSpec((_R, 16), lambda i: (i, 0)),
          pl.BlockSpec((_R, D), lambda i: (i, 0)),
          pl.BlockSpec((D, D), lambda i: (0, 0)),
          pl.BlockSpec((1, D), lambda i: (0, 0)),
          pl.BlockSpec((D, D), lambda i: (0, 0)),
      ],
      out_specs=out_specs,
      out_shape=out_shape,
  )


_dense_relu_split = _make_dense(True, True)
_dense_lin = _make_dense(False, False)


def kernel(x, edge_index, W1_l, b1_l, W1_r, W2_l, b2_l, W2_r):
  src = edge_index[0].astype(jnp.int32).reshape(NS, NCH, CHUNK)
  dst = edge_index[1].astype(jnp.int32).reshape(NS, NCH, CHUNK)
  xs = jnp.moveaxis(x.reshape(N, NC, DH), 1, 0)
  zrow = jnp.zeros((RPS, DH), jnp.float32)
  zcnt = jnp.zeros((RPS, 16), jnp.float32)
  ones = jnp.ones((CHUNK, 16), jnp.float32)
  b1 = b1_l.reshape(1, D)
  b2 = b2_l.reshape(1, D)

  agg1, cnt = _agg_counts(xs, src, dst, zrow, zcnt, ones)
  h, hs = _dense_relu_split(agg1, cnt, x, W1_l, b1, W1_r)
  agg2, = _agg_plain(hs, src, dst, zrow)
  out, = _dense_lin(agg2, cnt, h, W2_l, b2, W2_r)
  return out


# speedup vs baseline: 11.1296x; 1.2377x over previous
import functools

import jax
import jax.numpy as jnp
from jax import lax
from jax.experimental import pallas as pl
from jax.experimental.pallas import tpu as pltpu
from jax.experimental.pallas import tpu_sc as plsc

N = 10000
E = 320000
D = 128
DH = D // 2

NC = 2
NS = 16
EPW = E // NS
CHUNK = 125
NCH = EPW // CHUNK
NBUF = 4
NG = NCH // NBUF
RPAD = 10240
RPS = RPAD // NS

_mesh = plsc.VectorSubcoreMesh(core_axis_name="c", subcore_axis_name="s")


def _make_agg(with_counts: bool):
  out_type = [jax.ShapeDtypeStruct((NC, RPAD, DH), jnp.float32)]
  scratch = [
      pltpu.VMEM((NCH, CHUNK), jnp.int32),
      pltpu.VMEM((NCH, CHUNK), jnp.int32),
      [pltpu.VMEM((CHUNK, DH), jnp.float32) for _ in range(NBUF)],
      pltpu.VMEM_SHARED((RPAD, DH), jnp.float32),
      [pltpu.SemaphoreType.DMA for _ in range(NBUF)],
      [pltpu.SemaphoreType.DMA for _ in range(NBUF)],
  ]
  if with_counts:
    out_type.append(jax.ShapeDtypeStruct((NC, RPAD, 16), jnp.float32))
    scratch.append(pltpu.VMEM((CHUNK, 16), jnp.float32))
    scratch.append(pltpu.VMEM_SHARED((RPAD, 16), jnp.float32))
    scratch.append(pltpu.SemaphoreType.DMA)

  @functools.partial(pl.kernel, mesh=_mesh, out_type=out_type,
                     scratch_types=scratch,
                     compiler_params=pltpu.CompilerParams(
                         use_tc_tiling_on_sc=False))
  def agg(*refs):
    if with_counts:
      (feat2, src3, dst3, zrow, zcnt, ones_h,
       agg_out, cnt_out, src_v, dst_v, rows, acc_s, gsem, ssem,
       ones_v, cnt_s, csem) = refs
    else:
      (feat2, src3, dst3, zrow,
       agg_out, src_v, dst_v, rows, acc_s, gsem, ssem) = refs
    cid = lax.axis_index("c")
    sid = lax.axis_index("s")
    row0 = sid * RPS
    pltpu.sync_copy(zrow, acc_s.at[pl.ds(row0, RPS)])
    pltpu.sync_copy(src3.at[sid], src_v)
    pltpu.sync_copy(dst3.at[sid], dst_v)
    if with_counts:
      pltpu.sync_copy(zcnt, cnt_s.at[pl.ds(row0, RPS)])
      pltpu.sync_copy(ones_h, ones_v)
    plsc.subcore_barrier()

    def gather(i, b):
      pltpu.async_copy(feat2.at[cid].at[src_v.at[i]], rows[b], gsem[b])

    def gwait(i, b):
      pltpu.make_async_copy(feat2.at[cid].at[src_v.at[i]], rows[b],
                            gsem[b]).wait()

    def swait(i, b):
      pltpu.make_async_copy(rows[b], acc_s.at[dst_v.at[i]], ssem[b]).wait()

    for b in range(NBUF):
      gather(b, b)

    def body(j, carry):
      i0 = NBUF * j
      for b in range(NBUF):
        i = i0 + b
        gwait(i, b)
        pltpu.async_copy(rows[b], acc_s.at[dst_v.at[i]], ssem[b], add=True)
        if with_counts:
          @pl.when(cid == b % 2)
          def _():
            pltpu.async_copy(ones_v, cnt_s.at[dst_v.at[i]], csem, add=True)
      for b in range(NBUF):
        i = i0 + b

        @pl.when(j < NG - 1)
        def _():
          swait(i, b)
          gather(i + NBUF, b)
      return carry

    lax.fori_loop(0, NG, body, 0)
    for b in range(NBUF):
      swait(NCH - NBUF + b, b)
    if with_counts:
      def cdrain(_, carry):
        pltpu.make_async_copy(ones_v, cnt_s.at[dst_v.at[0]], csem).wait()
        return carry
      lax.fori_loop(0, NCH // 2, cdrain, 0)
    plsc.subcore_barrier()
    pltpu.sync_copy(acc_s.at[pl.ds(row0, RPS)],
                    agg_out.at[cid, pl.ds(row0, RPS)])
    if with_counts:
      pltpu.sync_copy(cnt_s.at[pl.ds(row0, RPS)],
                      cnt_out.at[cid, pl.ds(row0, RPS)])

  return agg


_agg_counts = _make_agg(True)
_agg_plain = _make_agg(False)

_R = 1000


def _make_dense(relu: bool, split_out: bool):
  def body(agg_ref, cnt_ref, x_ref, wl_ref, b_ref, wr_ref, *o_refs):
    cnt = cnt_ref[0, :, 0:1] + cnt_ref[1, :, 0:1]
    inv = 1.0 / jnp.maximum(cnt, 1.0)
    y = (jnp.dot(agg_ref[0] * inv, wl_ref[0:DH, :],
                 preferred_element_type=jnp.float32)
         + jnp.dot(agg_ref[1] * inv, wl_ref[DH:D, :],
                   preferred_element_type=jnp.float32)
         + b_ref[...]
         + jnp.dot(x_ref[...], wr_ref[...], preferred_element_type=jnp.float32))
    if relu:
      y = jnp.maximum(y, 0.0)
    o_refs[0][...] = y
    if split_out:
      o_refs[1][0] = y[:, 0:DH]
      o_refs[1][1] = y[:, DH:D]

  out_shape = [jax.ShapeDtypeStruct((N, D), jnp.float32)]
  out_specs = [pl.BlockSpec((_R, D), lambda i: (i, 0))]
  if split_out:
    out_shape.append(jax.ShapeDtypeStruct((NC, N, DH), jnp.float32))
    out_specs.append(pl.BlockSpec((NC, _R, DH), lambda i: (0, i, 0)))

  return pl.pallas_call(
      body,
      grid=(N // _R,),
      in_specs=[
          pl.BlockSpec((NC, _R, DH), lambda i: (0, i, 0)),
          pl.BlockSpec((NC, _R, 16), lambda i: (0, i, 0)),
          pl.BlockSpec((_R, D), lambda i: (i, 0)),
          pl.BlockSpec((D, D), lambda i: (0, 0)),
          pl.BlockSpec((1, D), lambda i: (0, 0)),
          pl.BlockSpec((D, D), lambda i: (0, 0)),
      ],
      out_specs=out_specs,
      out_shape=out_shape,
  )


_dense_relu_split = _make_dense(True, True)
_dense_lin = _make_dense(False, False)


def kernel(x, edge_index, W1_l, b1_l, W1_r, W2_l, b2_l, W2_r):
  src = edge_index[0].astype(jnp.int32).reshape(NS, NCH, CHUNK)
  dst = edge_index[1].astype(jnp.int32).reshape(NS, NCH, CHUNK)
  xs = jnp.moveaxis(x.reshape(N, NC, DH), 1, 0)
  zrow = jnp.zeros((RPS, DH), jnp.float32)
  zcnt = jnp.zeros((RPS, 16), jnp.float32)
  ones = jnp.ones((CHUNK, 16), jnp.float32)
  b1 = b1_l.reshape(1, D)
  b2 = b2_l.reshape(1, D)

  agg1, cnt = _agg_counts(xs, src, dst, zrow, zcnt, ones)
  h, hs = _dense_relu_split(agg1, cnt, x, W1_l, b1, W1_r)
  agg2, = _agg_plain(hs, src, dst, zrow)
  out, = _dense_lin(agg2, cnt, h, W2_l, b2, W2_r)
  return out
